# baseline (speedup 1.0000x reference)
import math

import jax
import jax.numpy as jnp
from jax import lax
from jax.experimental import pallas as pl
from jax.experimental.pallas import tpu as pltpu
from jax.experimental.pallas import tpu_sc as plsc

N_NODES = 100000
N_EDGES = 1600000
N_HEADS = 8
HEAD_DIM = 4
FDIM = N_HEADS * HEAD_DIM
REC = FDIM + N_HEADS

NC = 2
NS = 16
N_HALF = N_NODES // NC
ACC_ROWS = 50000
ZCHUNK = ACC_ROWS // NS
SENT = -1

EB = 16
EDGES_PER_TILE = N_EDGES // NS
NBLOCKS = EDGES_PER_TILE // EB
SUP = 10
SUPE = SUP * EB
NSUP = NBLOCKS // SUP
PAIRS = SUP // 2 - 1

OUT_GROUPS = N_HALF // 16
INV_SQRT = 1.0 / math.sqrt(FDIM)


def _body(kv_hbm, q_hbm, dst_hbm, zeros_hbm, out_hbm,
          dsup0, db0, db1, ib0, ib1, q0, q1, kv0, kv1,
          r0, r1, acc, sl0, sl1, ss0, ss1):
    dstb, idxb = [db0, db1], [ib0, ib1]
    qb, kvb, rb = [q0, q1], [kv0, kv1], [r0, r1]
    sl, ss = [sl0, sl1], [ss0, ss1]

    cid = lax.axis_index("c")
    sid = lax.axis_index("s")
    node_base = cid * N_HALF
    tile_base = sid * EDGES_PER_TILE
    rows16 = lax.iota(jnp.int32, 16)

    pltpu.sync_copy(zeros_hbm, acc.at[pl.ds(sid * ZCHUNK, ZCHUNK)])
    plsc.subcore_barrier()

    def issue_loads(goff, dst16, b):
        oob = (dst16 < node_base) | (dst16 >= node_base + N_HALF)
        dstb[b][...] = jnp.where(oob, SENT, dst16)
        pltpu.async_copy(kv_hbm.at[:, pl.ds(goff, EB)], kvb[b], sl[b])
        pltpu.async_copy(
            q_hbm.at[plsc.Indices(dstb[b], ignored_value=SENT)], qb[b], sl[b])

    def proc_block(goff, b):
        pltpu.make_async_copy(
            kv_hbm.at[:, pl.ds(goff, EB)], kvb[b], sl[b]).wait()
        pltpu.make_async_copy(
            q_hbm.at[plsc.Indices(dstb[b], ignored_value=SENT)],
            qb[b], sl[b]).wait()
        d16 = dstb[b][...]
        loc = d16 - node_base
        idx = jnp.where(d16 == SENT, SENT, loc)
        pltpu.make_async_copy(
            rb[b], acc.at[plsc.Indices(idxb[b], ignored_value=SENT)],
            ss[b]).wait()
        idxb[b][...] = idx

        def head(h, carry):
            hb = h * HEAD_DIM
            acc_e = None
            for t in range(HEAD_DIM):
                kf = kvb[b][hb + t, :]
                qf = plsc.load_gather(
                    qb[b], [rows16, lax.broadcast(hb + t, (16,))])
                acc_e = kf * qf if acc_e is None else acc_e + kf * qf
            s = jnp.exp(acc_e * INV_SQRT)
            plsc.store_scatter(
                rb[b], [rows16, lax.broadcast(FDIM + h, (16,))], s)
            for t in range(HEAD_DIM):
                vf = kvb[b][FDIM + hb + t, :]
                plsc.store_scatter(
                    rb[b], [rows16, lax.broadcast(hb + t, (16,))], s * vf)
            return carry

        lax.fori_loop(0, N_HEADS, head, 0)
        pltpu.async_copy(
            rb[b], acc.at[plsc.Indices(idxb[b], ignored_value=SENT)],
            ss[b], add=True)

    def super_chunk(s_val, is_last):
        sbase = tile_base + s_val * SUPE

        def pair(j, carry):
            for b in (0, 1):
                goff = sbase + j * 32 + b * EB
                proc_block(goff, b)
                d16n = dsup0[pl.ds((j + 1) * 32 + b * EB, EB)]
                issue_loads(goff + 32, d16n, b)
            return carry

        lax.fori_loop(0, PAIRS, pair, 0)
        if not is_last:
            pltpu.sync_copy(dst_hbm.at[pl.ds(sbase + SUPE, SUPE)], dsup0)
        jlast = PAIRS
        for b in (0, 1):
            goff = sbase + jlast * 32 + b * EB
            proc_block(goff, b)
            if not is_last:
                d16n = dsup0[pl.ds(b * EB, EB)]
                issue_loads(sbase + SUPE + b * EB, d16n, b)

    for b in (0, 1):
        idxb[b][...] = jnp.full((EB,), SENT, jnp.int32)
        pltpu.async_copy(
            rb[b], acc.at[plsc.Indices(idxb[b], ignored_value=SENT)],
            ss[b], add=True)
    pltpu.sync_copy(dst_hbm.at[pl.ds(tile_base, SUPE)], dsup0)
    for b in (0, 1):
        issue_loads(tile_base + b * EB, dsup0[pl.ds(b * EB, EB)], b)

    def one_super(t, carry):
        super_chunk(t, False)
        return carry

    lax.fori_loop(0, NSUP - 1, one_super, 0)
    super_chunk(NSUP - 1, True)

    for b in (0, 1):
        pltpu.make_async_copy(
            rb[b], acc.at[plsc.Indices(idxb[b], ignored_value=SENT)],
            ss[b]).wait()
    plsc.subcore_barrier()

    ngroups = 195 + jnp.where(sid < OUT_GROUPS - 195 * NS, 1, 0)
    tmp_v = rb[0]
    o16_v = qb[0]

    def out_group(i, carry):
        r = (sid + NS * i) * 16
        pltpu.sync_copy(acc.at[pl.ds(r, 16)], tmp_v)
        one = jnp.full((16,), 1, jnp.int32)

        def ohead(h, carry):
            den = plsc.load_gather(
                tmp_v, [rows16, lax.broadcast(FDIM + h, (16,))]) + 1e-9
            cn = lax.broadcast(h * HEAD_DIM, (16,))
            for t in range(HEAD_DIM):
                num = plsc.load_gather(tmp_v, [rows16, cn])
                plsc.store_scatter(o16_v, [rows16, cn], num / den)
                cn = cn + one
            return carry

        lax.fori_loop(0, N_HEADS, ohead, 0)
        pltpu.sync_copy(o16_v, out_hbm.at[pl.ds(node_base + r, 16)])
        return carry

    lax.fori_loop(0, ngroups, out_group, 0)


@jax.jit
def kernel(v, k, q, edge_index):
    kv = jnp.concatenate(
        [k.transpose(), v.reshape(N_EDGES, FDIM).transpose()], axis=0)
    dst = edge_index[1].astype(jnp.int32)
    zeros = jnp.zeros((ZCHUNK, REC), jnp.float32)
    mesh = plsc.VectorSubcoreMesh(
        core_axis_name="c", subcore_axis_name="s",
        num_cores=NC, num_subcores=NS)
    fn = pl.kernel(
        _body,
        out_type=jax.ShapeDtypeStruct((N_NODES, FDIM), jnp.float32),
        mesh=mesh,
        compiler_params=pltpu.CompilerParams(
            needs_layout_passes=False, use_tc_tiling_on_sc=False),
        scratch_types=[
            pltpu.VMEM((SUPE,), jnp.int32),
            pltpu.VMEM((EB,), jnp.int32),
            pltpu.VMEM((EB,), jnp.int32),
            pltpu.VMEM((EB,), jnp.int32),
            pltpu.VMEM((EB,), jnp.int32),
            pltpu.VMEM((EB, FDIM), jnp.float32),
            pltpu.VMEM((EB, FDIM), jnp.float32),
            pltpu.VMEM((2 * FDIM, EB), jnp.float32),
            pltpu.VMEM((2 * FDIM, EB), jnp.float32),
            pltpu.VMEM((EB, REC), jnp.float32),
            pltpu.VMEM((EB, REC), jnp.float32),
            pltpu.VMEM_SHARED((ACC_ROWS, REC), jnp.float32),
            pltpu.SemaphoreType.DMA,
            pltpu.SemaphoreType.DMA,
            pltpu.SemaphoreType.DMA,
            pltpu.SemaphoreType.DMA,
        ],
    )
    out = fn(kv, q, dst, zeros)
    return out.reshape(N_NODES, FDIM, 1)


# speedup vs baseline: 1.4306x; 1.4306x over previous
import math

import jax
import jax.numpy as jnp
from jax import lax
from jax.experimental import pallas as pl
from jax.experimental.pallas import tpu as pltpu
from jax.experimental.pallas import tpu_sc as plsc

N_NODES = 100000
N_EDGES = 1600000
N_HEADS = 8
HEAD_DIM = 4
FDIM = N_HEADS * HEAD_DIM
REC = FDIM + N_HEADS

NC = 2
NS = 16
N_HALF = N_NODES // NC
ACC_ROWS = 50000
ZCHUNK = ACC_ROWS // NS
SENT = -1

EB = 16
EDGES_PER_TILE = N_EDGES // NS
NBLOCKS = EDGES_PER_TILE // EB
SUP = 10
SUPE = SUP * EB
NSUP = NBLOCKS // SUP
PAIRS = SUP // 2 - 1

OUT_GROUPS = N_HALF // 16
INV_SQRT = 1.0 / math.sqrt(FDIM)


def _body(kv_hbm, q_hbm, dst_hbm, zeros_hbm, out_hbm,
          dsup0, db0, db1, ib0, ib1, q0, q1, kv0, kv1,
          r0, r1, acc, sl0, sl1, ss0, ss1):
    dstb, idxb = [db0, db1], [ib0, ib1]
    qb, kvb, rb = [q0, q1], [kv0, kv1], [r0, r1]
    sl, ss = [sl0, sl1], [ss0, ss1]

    cid = lax.axis_index("c")
    sid = lax.axis_index("s")
    node_base = cid * N_HALF
    tile_base = sid * EDGES_PER_TILE
    rows16 = lax.iota(jnp.int32, 16)

    pltpu.sync_copy(zeros_hbm, acc.at[pl.ds(sid * ZCHUNK, ZCHUNK)])
    plsc.subcore_barrier()

    def issue_loads(goff, dst16, b):
        oob = (dst16 < node_base) | (dst16 >= node_base + N_HALF)
        dstb[b][...] = jnp.where(oob, SENT, dst16)
        pltpu.async_copy(kv_hbm.at[pl.ds(goff, EB)], kvb[b], sl[b])
        pltpu.async_copy(
            q_hbm.at[plsc.Indices(dstb[b], ignored_value=SENT)], qb[b], sl[b])

    def proc_block(goff, b):
        pltpu.make_async_copy(kv_hbm.at[pl.ds(goff, EB)], kvb[b], sl[b]).wait()
        pltpu.make_async_copy(
            q_hbm.at[plsc.Indices(dstb[b], ignored_value=SENT)],
            qb[b], sl[b]).wait()
        d16 = dstb[b][...]
        loc = d16 - node_base
        idx = jnp.where(d16 == SENT, SENT, loc)
        pltpu.make_async_copy(
            rb[b], acc.at[plsc.Indices(idxb[b], ignored_value=SENT)],
            ss[b]).wait()
        idxb[b][...] = idx

        def head2(hh, carry):
            for u in range(2):
                h = hh * 2 + u
                hb = h * HEAD_DIM
                acc_e = None
                for t in range(HEAD_DIM):
                    c = lax.broadcast(hb + t, (16,))
                    kf = plsc.load_gather(kvb[b], [rows16, c])
                    qf = plsc.load_gather(qb[b], [rows16, c])
                    p = kf * qf
                    acc_e = p if acc_e is None else acc_e + p
                s = jnp.exp(acc_e * INV_SQRT)
                plsc.store_scatter(
                    rb[b], [rows16, lax.broadcast(FDIM + h, (16,))], s)
                for t in range(HEAD_DIM):
                    c = lax.broadcast(hb + t, (16,))
                    vf = plsc.load_gather(
                        kvb[b], [rows16, lax.broadcast(FDIM + hb + t, (16,))])
                    plsc.store_scatter(rb[b], [rows16, c], s * vf)
            return carry

        lax.fori_loop(0, N_HEADS // 2, head2, 0)
        pltpu.async_copy(
            rb[b], acc.at[plsc.Indices(idxb[b], ignored_value=SENT)],
            ss[b], add=True)

    def super_chunk(s_val, is_last):
        sbase = tile_base + s_val * SUPE

        def pair(j, carry):
            for b in (0, 1):
                goff = sbase + j * 32 + b * EB
                proc_block(goff, b)
                d16n = dsup0[pl.ds((j + 1) * 32 + b * EB, EB)]
                issue_loads(goff + 32, d16n, b)
            return carry

        lax.fori_loop(0, PAIRS, pair, 0)
        if not is_last:
            pltpu.sync_copy(dst_hbm.at[pl.ds(sbase + SUPE, SUPE)], dsup0)
        jlast = PAIRS
        for b in (0, 1):
            goff = sbase + jlast * 32 + b * EB
            proc_block(goff, b)
            if not is_last:
                d16n = dsup0[pl.ds(b * EB, EB)]
                issue_loads(sbase + SUPE + b * EB, d16n, b)

    for b in (0, 1):
        idxb[b][...] = jnp.full((EB,), SENT, jnp.int32)
        pltpu.async_copy(
            rb[b], acc.at[plsc.Indices(idxb[b], ignored_value=SENT)],
            ss[b], add=True)
    pltpu.sync_copy(dst_hbm.at[pl.ds(tile_base, SUPE)], dsup0)
    for b in (0, 1):
        issue_loads(tile_base + b * EB, dsup0[pl.ds(b * EB, EB)], b)

    def one_super(t, carry):
        super_chunk(t, False)
        return carry

    lax.fori_loop(0, NSUP - 1, one_super, 0)
    super_chunk(NSUP - 1, True)

    for b in (0, 1):
        pltpu.make_async_copy(
            rb[b], acc.at[plsc.Indices(idxb[b], ignored_value=SENT)],
            ss[b]).wait()
    plsc.subcore_barrier()

    ngroups = 195 + jnp.where(sid < OUT_GROUPS - 195 * NS, 1, 0)
    tmp_v = rb[0]
    o16_v = qb[0]

    def out_group(i, carry):
        r = (sid + NS * i) * 16
        pltpu.sync_copy(acc.at[pl.ds(r, 16)], tmp_v)
        one = jnp.full((16,), 1, jnp.int32)

        def ohead(h, carry):
            den = plsc.load_gather(
                tmp_v, [rows16, lax.broadcast(FDIM + h, (16,))]) + 1e-9
            cn = lax.broadcast(h * HEAD_DIM, (16,))
            for t in range(HEAD_DIM):
                num = plsc.load_gather(tmp_v, [rows16, cn])
                plsc.store_scatter(o16_v, [rows16, cn], num / den)
                cn = cn + one
            return carry

        lax.fori_loop(0, N_HEADS, ohead, 0)
        pltpu.sync_copy(o16_v, out_hbm.at[pl.ds(node_base + r, 16)])
        return carry

    lax.fori_loop(0, ngroups, out_group, 0)


@jax.jit
def kernel(v, k, q, edge_index):
    kv = jnp.concatenate([k, v.reshape(N_EDGES, FDIM)], axis=1)
    dst = edge_index[1].astype(jnp.int32)
    zeros = jnp.zeros((ZCHUNK, REC), jnp.float32)
    mesh = plsc.VectorSubcoreMesh(
        core_axis_name="c", subcore_axis_name="s",
        num_cores=NC, num_subcores=NS)
    fn = pl.kernel(
        _body,
        out_type=jax.ShapeDtypeStruct((N_NODES, FDIM), jnp.float32),
        mesh=mesh,
        compiler_params=pltpu.CompilerParams(
            needs_layout_passes=False, use_tc_tiling_on_sc=False),
        scratch_types=[
            pltpu.VMEM((SUPE,), jnp.int32),
            pltpu.VMEM((EB,), jnp.int32),
            pltpu.VMEM((EB,), jnp.int32),
            pltpu.VMEM((EB,), jnp.int32),
            pltpu.VMEM((EB,), jnp.int32),
            pltpu.VMEM((EB, FDIM), jnp.float32),
            pltpu.VMEM((EB, FDIM), jnp.float32),
            pltpu.VMEM((EB, 2 * FDIM), jnp.float32),
            pltpu.VMEM((EB, 2 * FDIM), jnp.float32),
            pltpu.VMEM((EB, REC), jnp.float32),
            pltpu.VMEM((EB, REC), jnp.float32),
            pltpu.VMEM_SHARED((ACC_ROWS, REC), jnp.float32),
            pltpu.SemaphoreType.DMA,
            pltpu.SemaphoreType.DMA,
            pltpu.SemaphoreType.DMA,
            pltpu.SemaphoreType.DMA,
        ],
    )
    out = fn(kv, q, dst, zeros)
    return out.reshape(N_NODES, FDIM, 1)


# speedup vs baseline: 1.5000x; 1.0485x over previous
import math

import jax
import jax.numpy as jnp
from jax import lax
from jax.experimental import pallas as pl
from jax.experimental.pallas import tpu as pltpu
from jax.experimental.pallas import tpu_sc as plsc

N_NODES = 100000
N_EDGES = 1600000
N_HEADS = 8
HEAD_DIM = 4
FDIM = N_HEADS * HEAD_DIM
REC = FDIM + N_HEADS

NC = 2
NS = 16
N_HALF = N_NODES // NC
ACC_ROWS = 50016
DUMP_ROW = ACC_ROWS - 1
ZCHUNK = ACC_ROWS // NS

EB = 16
EDGES_PER_TILE = N_EDGES // NS
NBLOCKS = EDGES_PER_TILE // EB
SUP = 10
SUPE = SUP * EB
NSUP = NBLOCKS // SUP
PAIRS = SUP // 2 - 1

OUT_GROUPS = N_HALF // 16
INV_SQRT = 1.0 / math.sqrt(FDIM)


def _body(k_hbm, q_hbm, v_hbm, dst_hbm, zeros_hbm, out_hbm,
          dsup0, dsup1, db0, db1, ib0, ib1, q0, q1, k0, k1, v0, v1,
          r0, r1, acc, sd0, sd1, sl0, sl1, ss0, ss1):
    dsup = [dsup0, dsup1]
    dstb, idxb = [db0, db1], [ib0, ib1]
    qb, kb, vb, rb = [q0, q1], [k0, k1], [v0, v1], [r0, r1]
    sd, sl, ss = [sd0, sd1], [sl0, sl1], [ss0, ss1]

    cid = lax.axis_index("c")
    sid = lax.axis_index("s")
    node_base = cid * N_HALF
    tile_base = sid * EDGES_PER_TILE
    rows16 = lax.iota(jnp.int32, 16)

    pltpu.sync_copy(zeros_hbm, acc.at[pl.ds(sid * ZCHUNK, ZCHUNK)])
    plsc.subcore_barrier()

    def issue_loads(goff, dst16, b):
        dstb[b][...] = dst16
        pltpu.async_copy(k_hbm.at[pl.ds(goff, EB)], kb[b], sl[b])
        pltpu.async_copy(v_hbm.at[pl.ds(goff, EB)], vb[b], sl[b])
        pltpu.async_copy(q_hbm.at[dstb[b]], qb[b], sl[b])

    def proc_block(goff, b, first):
        pltpu.make_async_copy(k_hbm.at[pl.ds(goff, EB)], kb[b], sl[b]).wait()
        pltpu.make_async_copy(v_hbm.at[pl.ds(goff, EB)], vb[b], sl[b]).wait()
        pltpu.make_async_copy(q_hbm.at[dstb[b]], qb[b], sl[b]).wait()
        loc = dstb[b][...] - node_base
        oob = (loc < 0) | (loc >= N_HALF)
        idx = jnp.where(oob, DUMP_ROW, loc)

        @pl.when(jnp.logical_not(first))
        def _():
            pltpu.make_async_copy(rb[b], acc.at[idxb[b]], ss[b]).wait()

        idxb[b][...] = idx
        for h in range(N_HEADS):
            acc_e = None
            for t in range(HEAD_DIM):
                col = jnp.full((16,), h * HEAD_DIM + t, jnp.int32)
                kf = plsc.load_gather(kb[b], [rows16, col])
                qf = plsc.load_gather(qb[b], [rows16, col])
                acc_e = kf * qf if acc_e is None else acc_e + kf * qf
            s = jnp.exp(acc_e * INV_SQRT)
            plsc.store_scatter(
                rb[b], [rows16, jnp.full((16,), FDIM + h, jnp.int32)], s)
            for t in range(HEAD_DIM):
                col = jnp.full((16,), h * HEAD_DIM + t, jnp.int32)
                vf = plsc.load_gather(vb[b], [rows16, col])
                plsc.store_scatter(rb[b], [rows16, col], s * vf)
        pltpu.async_copy(rb[b], acc.at[idxb[b]], ss[b], add=True)

    def super_chunk(s_val, par, is_last):
        dcur, dnxt = dsup[par], dsup[1 - par]
        sbase = tile_base + s_val * SUPE
        if not is_last:
            pltpu.async_copy(
                dst_hbm.at[pl.ds(sbase + SUPE, SUPE)], dnxt, sd[1 - par])

        def pair(j, carry):
            for b in (0, 1):
                goff = sbase + j * 32 + b * EB
                proc_block(goff, b, (s_val == 0) & (j == 0))
                d16n = dcur[pl.ds((j + 1) * 32 + b * EB, EB)]
                issue_loads(goff + 32, d16n, b)
            return carry

        lax.fori_loop(0, PAIRS, pair, 0)
        if not is_last:
            pltpu.make_async_copy(
                dst_hbm.at[pl.ds(sbase + SUPE, SUPE)], dnxt, sd[1 - par]).wait()
        jlast = PAIRS
        for b in (0, 1):
            goff = sbase + jlast * 32 + b * EB
            proc_block(goff, b, False)
            if not is_last:
                d16n = dnxt[pl.ds(b * EB, EB)]
                issue_loads(sbase + SUPE + b * EB, d16n, b)

    pltpu.sync_copy(dst_hbm.at[pl.ds(tile_base, SUPE)], dsup[0])
    for b in (0, 1):
        issue_loads(tile_base + b * EB, dsup[0][pl.ds(b * EB, EB)], b)

    def two_supers(t, carry):
        super_chunk(2 * t, 0, False)
        super_chunk(2 * t + 1, 1, False)
        return carry

    lax.fori_loop(0, (NSUP - 1) // 2, two_supers, 0)
    super_chunk(NSUP - 1, 0, True)

    for b in (0, 1):
        pltpu.make_async_copy(rb[b], acc.at[idxb[b]], ss[b]).wait()
    plsc.subcore_barrier()

    ngroups = 195 + jnp.where(sid < OUT_GROUPS - 195 * NS, 1, 0)
    tmp_v = rb[0]
    o16_v = qb[0]

    def out_group(i, carry):
        r = (sid + NS * i) * 16
        pltpu.sync_copy(acc.at[pl.ds(r, 16)], tmp_v)
        for h in range(N_HEADS):
            den = plsc.load_gather(
                tmp_v, [rows16, jnp.full((16,), FDIM + h, jnp.int32)]) + 1e-9
            for t in range(HEAD_DIM):
                col = jnp.full((16,), h * HEAD_DIM + t, jnp.int32)
                num = plsc.load_gather(tmp_v, [rows16, col])
                plsc.store_scatter(o16_v, [rows16, col], num / den)
        pltpu.sync_copy(o16_v, out_hbm.at[pl.ds(node_base + r, 16)])
        return carry

    lax.fori_loop(0, ngroups, out_group, 0)


@jax.jit
def kernel(v, k, q, edge_index):
    v2 = v.reshape(N_EDGES, FDIM)
    dst = edge_index[1].astype(jnp.int32)
    zeros = jnp.zeros((ZCHUNK, REC), jnp.float32)
    mesh = plsc.VectorSubcoreMesh(
        core_axis_name="c", subcore_axis_name="s",
        num_cores=NC, num_subcores=NS)
    fn = pl.kernel(
        _body,
        out_type=jax.ShapeDtypeStruct((N_NODES, FDIM), jnp.float32),
        mesh=mesh,
        compiler_params=pltpu.CompilerParams(
            needs_layout_passes=False, use_tc_tiling_on_sc=False),
        scratch_types=[
            pltpu.VMEM((SUPE,), jnp.int32),
            pltpu.VMEM((SUPE,), jnp.int32),
            pltpu.VMEM((EB,), jnp.int32),
            pltpu.VMEM((EB,), jnp.int32),
            pltpu.VMEM((EB,), jnp.int32),
            pltpu.VMEM((EB,), jnp.int32),
            pltpu.VMEM((EB, FDIM), jnp.float32),
            pltpu.VMEM((EB, FDIM), jnp.float32),
            pltpu.VMEM((EB, FDIM), jnp.float32),
            pltpu.VMEM((EB, FDIM), jnp.float32),
            pltpu.VMEM((EB, FDIM), jnp.float32),
            pltpu.VMEM((EB, FDIM), jnp.float32),
            pltpu.VMEM((EB, REC), jnp.float32),
            pltpu.VMEM((EB, REC), jnp.float32),
            pltpu.VMEM_SHARED((ACC_ROWS, REC), jnp.float32),
            pltpu.SemaphoreType.DMA,
            pltpu.SemaphoreType.DMA,
            pltpu.SemaphoreType.DMA,
            pltpu.SemaphoreType.DMA,
            pltpu.SemaphoreType.DMA,
            pltpu.SemaphoreType.DMA,
        ],
    )
    out = fn(k, q, v2, dst, zeros)
    return out.reshape(N_NODES, FDIM, 1)
